# baseline (speedup 1.0000x reference)
import jax
import jax.numpy as jnp
from jax.experimental import pallas as pl
from jax.experimental.pallas import tpu as pltpu

NUM_EXPERTS = 16
TOP_K = 2
ROW_TILE = 1024


def _indices(y):
    bits = (y > 0).astype(jnp.int32)
    s0 = jnp.sum(bits[:32, :], axis=0, keepdims=True)
    s1 = jnp.sum(bits[32:, :], axis=0, keepdims=True)
    r0 = jnp.bitwise_and(s0, NUM_EXPERTS - 1)
    r1 = jnp.bitwise_and(s1, NUM_EXPERTS - 1)
    i1 = jnp.where(r1 == r0, jnp.bitwise_and(r0 + 1, NUM_EXPERTS - 1), r1)
    return jnp.concatenate([r0, i1], axis=0)


def _router_body(xa_ref, xb_ref, w_ref, b_ref, ia_ref, ib_ref):
    dims = (((1,), (1,)), ((), ()))
    ya = jax.lax.dot_general(w_ref[...], xa_ref[0], dims,
                             preferred_element_type=jnp.float32) + b_ref[...]
    ia_ref[...] = _indices(ya)
    yb = jax.lax.dot_general(w_ref[...], xb_ref[0], dims,
                             preferred_element_type=jnp.float32) + b_ref[...]
    ib_ref[...] = _indices(yb)


def kernel(hidden_states, W, b):
    B, S, H = hidden_states.shape
    T = B * S
    n_tiles = T // ROW_TILE
    steps = n_tiles // 2
    x = hidden_states.reshape(n_tiles, ROW_TILE, H)
    idx_a, idx_b = pl.pallas_call(
        _router_body,
        grid=(steps,),
        in_specs=[
            pl.BlockSpec((1, ROW_TILE, H), lambda i: (2 * i, 0, 0)),
            pl.BlockSpec((1, ROW_TILE, H), lambda i: (2 * i + 1, 0, 0)),
            pl.BlockSpec((TOP_K * 32, H), lambda i: (0, 0)),
            pl.BlockSpec((TOP_K * 32, 1), lambda i: (0, 0)),
        ],
        out_specs=[
            pl.BlockSpec((TOP_K, ROW_TILE), lambda i: (0, i)),
            pl.BlockSpec((TOP_K, ROW_TILE), lambda i: (0, i)),
        ],
        out_shape=[
            jax.ShapeDtypeStruct((TOP_K, steps * ROW_TILE), jnp.int32),
            jax.ShapeDtypeStruct((TOP_K, steps * ROW_TILE), jnp.int32),
        ],
        compiler_params=pltpu.CompilerParams(
            dimension_semantics=("arbitrary",),
        ),
    )(x, x, W, b.reshape(TOP_K * 32, 1))
    ia = idx_a.reshape(TOP_K, steps, ROW_TILE)
    ib = idx_b.reshape(TOP_K, steps, ROW_TILE)
    idx_t = jnp.stack([ia, ib], axis=2).reshape(TOP_K, T)
    expert_indices = idx_t.T.astype(jnp.int64)
    expert_weights = jnp.full((T, TOP_K), 1.0 / TOP_K, dtype=jnp.float32)
    router_logits = jnp.zeros((T, NUM_EXPERTS), dtype=jnp.float32)
    return (expert_weights, expert_indices, router_logits)


# speedup vs baseline: 1.1609x; 1.1609x over previous
import jax
import jax.numpy as jnp
from jax.experimental import pallas as pl
from jax.experimental.pallas import tpu as pltpu

NUM_EXPERTS = 16
TOP_K = 2
ROW_TILE = 1024


def _router_body(x_ref, w_ref, b_ref, idx_ref):
    y = jax.lax.dot_general(
        w_ref[...], x_ref[...],
        (((1,), (1,)), ((), ())),
        preferred_element_type=jnp.float32,
    )
    y = y + b_ref[...]
    bits = (y > 0).astype(jnp.int32)
    s0 = jnp.sum(bits[:32, :], axis=0, keepdims=True)
    s1 = jnp.sum(bits[32:, :], axis=0, keepdims=True)
    r0 = jnp.bitwise_and(s0, NUM_EXPERTS - 1)
    r1 = jnp.bitwise_and(s1, NUM_EXPERTS - 1)
    i1 = jnp.where(r1 == r0, jnp.bitwise_and(r0 + 1, NUM_EXPERTS - 1), r1)
    idx_ref[...] = jnp.concatenate([r0, i1], axis=0)


def kernel(hidden_states, W, b):
    B, S, H = hidden_states.shape
    T = B * S
    x = hidden_states.reshape(T, H)
    grid = (T // ROW_TILE,)
    idx_t = pl.pallas_call(
        _router_body,
        grid=grid,
        in_specs=[
            pl.BlockSpec((ROW_TILE, H), lambda i: (i, 0)),
            pl.BlockSpec((TOP_K * 32, H), lambda i: (0, 0)),
            pl.BlockSpec((TOP_K * 32, 1), lambda i: (0, 0)),
        ],
        out_specs=pl.BlockSpec((TOP_K, ROW_TILE), lambda i: (0, i)),
        out_shape=jax.ShapeDtypeStruct((TOP_K, T), jnp.int32),
        compiler_params=pltpu.CompilerParams(
            dimension_semantics=("parallel",),
        ),
    )(x, W, b.reshape(TOP_K * 32, 1))
    expert_indices = idx_t.T.astype(jnp.int64)
    expert_weights = jnp.full((T, TOP_K), 1.0 / TOP_K, dtype=jnp.float32)
    router_logits = jnp.zeros((T, NUM_EXPERTS), dtype=jnp.float32)
    return (expert_weights, expert_indices, router_logits)
